# baseline (speedup 1.0000x reference)
import functools

import jax
import jax.numpy as jnp
from jax import lax
from jax.experimental import pallas as pl
from jax.experimental.pallas import tpu as pltpu
from jax.experimental.pallas import tpu_sc as plsc

N_TOKENS = 16384
HIDDEN = 2048
N_EXP = 16
LANE = 16

N_SC = 1024
N_TC = N_TOKENS - N_SC
BLK = 512
NW = 32
TW = N_SC // NW
CH = 16
NCH = TW // CH


def _tc_gate_kernel(h_ref, w_ref, b_ref, w1_ref, w2_ref, i1_ref, i2_ref):
    h_blk = h_ref[...]
    w = w_ref[...]
    logits = lax.dot_general(
        w, h_blk, (((1,), (1,)), ((), ())),
        preferred_element_type=jnp.float32)
    logits = logits + b_ref[...]

    m1 = logits[0]
    i1 = jnp.zeros((BLK,), jnp.int32)
    m2 = jnp.full((BLK,), -jnp.inf)
    i2 = jnp.zeros((BLK,), jnp.int32)
    for e in range(1, N_EXP):
        x = logits[e]
        new_top = x > m1
        new_sec = jnp.logical_and(jnp.logical_not(new_top), x > m2)
        m2 = jnp.where(new_top, m1, jnp.where(new_sec, x, m2))
        i2 = jnp.where(new_top, i1, jnp.where(new_sec, e, i2))
        m1 = jnp.where(new_top, x, m1)
        i1 = jnp.where(new_top, e, i1)

    e2 = jnp.exp(m2 - m1)
    inv = 1.0 / (1.0 + e2)
    w1_ref[...] = inv
    w2_ref[...] = e2 * inv
    i1_ref[...] = i1
    i2_ref[...] = i2


_BITREV = (0, 8, 4, 12, 2, 10, 6, 14, 1, 9, 5, 13, 3, 11, 7, 15)
_GDN = lax.GatherDimensionNumbers(
    offset_dims=(), collapsed_slice_dims=(0,), start_index_map=(0,))


def _permute(x, idx):
    return lax.gather(x, idx[:, None], _GDN, (1,),
                      mode=lax.GatherScatterMode.PROMISE_IN_BOUNDS)


def _round_bf16(x):
    c = x * 65537.0
    return c - (c - x)


def _allmax(x, lanes):
    for s in (8, 4, 2, 1):
        x = jnp.maximum(x, _permute(x, lanes ^ s))
    return x


def _argeq(x, m, lanes):
    cand = jnp.where(x == m, lanes, LANE)
    for s in (8, 4, 2, 1):
        cand = jnp.minimum(cand, _permute(cand, lanes ^ s))
    return cand


def _sc_gate_kernel(h_hbm, w_hbm, b_hbm, w1_hbm, w2_hbm, i1_hbm, i2_hbm,
                    w_vmem, h_buf, ow1, ow2, oi1, oi2, b_vmem):
    wid = lax.axis_index("s") * 2 + lax.axis_index("c")
    base = N_TC + wid * TW

    pltpu.sync_copy(w_hbm, w_vmem)
    pltpu.sync_copy(b_hbm, b_vmem)
    bias = b_vmem[...]
    lanes = lax.iota(jnp.int32, LANE)
    zeros = jnp.zeros((LANE,), jnp.float32)
    izeros = jnp.zeros((LANE,), jnp.int32)

    def w_round_body(c, _):
        cc = c * LANE
        for e in range(N_EXP):
            w_vmem[e, pl.ds(cc, LANE)] = _round_bf16(w_vmem[e, pl.ds(cc, LANE)])
        return 0
    lax.fori_loop(0, HIDDEN // LANE, w_round_body, 0)

    def pair_body(p, accs4):
        w1a, w2a, i1a, i2a = accs4
        t0 = 2 * p

        def c_body(c, accs):
            cc = c * LANE
            a0, a1 = accs
            h0 = _round_bf16(h_buf[t0, pl.ds(cc, LANE)])
            h1 = _round_bf16(h_buf[t0 + 1, pl.ds(cc, LANE)])
            na0 = []
            na1 = []
            for e in range(N_EXP):
                wv = w_vmem[e, pl.ds(cc, LANE)]
                na0.append(a0[e] + h0 * wv)
                na1.append(a1[e] + h1 * wv)
            return (tuple(na0), tuple(na1))

        init = (tuple(zeros for _ in range(N_EXP)),
                tuple(zeros for _ in range(N_EXP)))
        accs = lax.fori_loop(0, HIDDEN // LANE, c_body, init)

        for tok in range(2):
            acc = accs[tok]
            vecs = [acc[_BITREV[i]] for i in range(N_EXP)]
            for s in (8, 4, 2, 1):
                sel = (lanes & s) == 0
                vecs = [jnp.where(sel,
                                  vecs[2 * i] + _permute(vecs[2 * i],
                                                         lanes ^ s),
                                  vecs[2 * i + 1] + _permute(vecs[2 * i + 1],
                                                             lanes ^ s))
                        for i in range(len(vecs) // 2)]
            lv = vecs[0] + bias
            m1 = _allmax(lv, lanes)
            i1 = _argeq(lv, m1, lanes)
            masked = jnp.where(lanes == i1, -jnp.inf, lv)
            m2 = _allmax(masked, lanes)
            i2 = _argeq(masked, m2, lanes)
            ev = jnp.exp(m2 - m1)
            w1v = 1.0 / (1.0 + ev)
            t = t0 + tok
            hit = lanes == t
            w1a = jnp.where(hit, w1v, w1a)
            w2a = jnp.where(hit, ev * w1v, w2a)
            i1a = jnp.where(hit, i1, i1a)
            i2a = jnp.where(hit, i2, i2a)
        return (w1a, w2a, i1a, i2a)

    for ch in range(NCH):
        pltpu.sync_copy(h_hbm.at[pl.ds(base + ch * CH, CH)], h_buf)
        w1a, w2a, i1a, i2a = lax.fori_loop(
            0, CH // 2, pair_body, (zeros, zeros, izeros, izeros))
        ow1[pl.ds(ch * CH, LANE)] = w1a
        ow2[pl.ds(ch * CH, LANE)] = w2a
        oi1[pl.ds(ch * CH, LANE)] = i1a
        oi2[pl.ds(ch * CH, LANE)] = i2a

    pltpu.sync_copy(ow1, w1_hbm.at[pl.ds(wid * TW, TW)])
    pltpu.sync_copy(ow2, w2_hbm.at[pl.ds(wid * TW, TW)])
    pltpu.sync_copy(oi1, i1_hbm.at[pl.ds(wid * TW, TW)])
    pltpu.sync_copy(oi2, i2_hbm.at[pl.ds(wid * TW, TW)])


_sc_gate = functools.partial(
    pl.kernel,
    out_type=[
        jax.ShapeDtypeStruct((N_SC,), jnp.float32),
        jax.ShapeDtypeStruct((N_SC,), jnp.float32),
        jax.ShapeDtypeStruct((N_SC,), jnp.int32),
        jax.ShapeDtypeStruct((N_SC,), jnp.int32),
    ],
    mesh=plsc.VectorSubcoreMesh(core_axis_name="c", subcore_axis_name="s",
                                num_cores=2, num_subcores=16),
    scratch_types=[
        pltpu.VMEM((N_EXP, HIDDEN), jnp.float32),
        pltpu.VMEM((CH, HIDDEN), jnp.float32),
        pltpu.VMEM((TW,), jnp.float32),
        pltpu.VMEM((TW,), jnp.float32),
        pltpu.VMEM((TW,), jnp.int32),
        pltpu.VMEM((TW,), jnp.int32),
        pltpu.VMEM((LANE,), jnp.float32),
    ],
)(_sc_gate_kernel)


@jax.jit
def kernel(h, weight, bias):
    tw1, tw2, ti1, ti2 = pl.pallas_call(
        _tc_gate_kernel,
        grid=(N_TOKENS // BLK,),
        in_specs=[
            pl.BlockSpec((BLK, HIDDEN), lambda i: (i, 0)),
            pl.BlockSpec((N_EXP, HIDDEN), lambda i: (0, 0)),
            pl.BlockSpec((N_EXP, 1), lambda i: (0, 0)),
        ],
        out_specs=[
            pl.BlockSpec((BLK,), lambda i: (i,)),
            pl.BlockSpec((BLK,), lambda i: (i,)),
            pl.BlockSpec((BLK,), lambda i: (i,)),
            pl.BlockSpec((BLK,), lambda i: (i,)),
        ],
        out_shape=[
            jax.ShapeDtypeStruct((N_TOKENS,), jnp.float32),
            jax.ShapeDtypeStruct((N_TOKENS,), jnp.float32),
            jax.ShapeDtypeStruct((N_TOKENS,), jnp.int32),
            jax.ShapeDtypeStruct((N_TOKENS,), jnp.int32),
        ],
    )(h, weight, bias.reshape(N_EXP, 1))

    tw = jnp.stack([tw1, tw2], axis=-1)
    ti = jnp.stack([ti1, ti2], axis=-1)
    return (tw, ti)


# speedup vs baseline: 1.1584x; 1.1584x over previous
import jax
import jax.numpy as jnp
from jax import lax
from jax.experimental import pallas as pl

N_TOKENS = 16384
HIDDEN = 2048
N_EXP = 16
BLK = 1024


def _gate_kernel(h_ref, w_ref, b_ref, w1_ref, w2_ref, i1_ref, i2_ref):
    h_blk = h_ref[...]
    w = w_ref[...]
    logits = lax.dot_general(
        w, h_blk, (((1,), (1,)), ((), ())),
        preferred_element_type=jnp.float32)
    logits = logits + b_ref[...]

    m1 = logits[0]
    i1 = jnp.zeros((BLK,), jnp.int32)
    m2 = jnp.full((BLK,), -jnp.inf)
    i2 = jnp.zeros((BLK,), jnp.int32)
    for e in range(1, N_EXP):
        x = logits[e]
        new_top = x > m1
        new_sec = jnp.logical_and(jnp.logical_not(new_top), x > m2)
        m2 = jnp.where(new_top, m1, jnp.where(new_sec, x, m2))
        i2 = jnp.where(new_top, i1, jnp.where(new_sec, e, i2))
        m1 = jnp.where(new_top, x, m1)
        i1 = jnp.where(new_top, e, i1)

    e2 = jnp.exp(m2 - m1)
    inv = 1.0 / (1.0 + e2)
    w1_ref[...] = inv
    w2_ref[...] = e2 * inv
    i1_ref[...] = i1
    i2_ref[...] = i2


@jax.jit
def kernel(h, weight, bias):
    tw1, tw2, ti1, ti2 = pl.pallas_call(
        _gate_kernel,
        grid=(N_TOKENS // BLK,),
        in_specs=[
            pl.BlockSpec((BLK, HIDDEN), lambda i: (i, 0)),
            pl.BlockSpec((N_EXP, HIDDEN), lambda i: (0, 0)),
            pl.BlockSpec((N_EXP, 1), lambda i: (0, 0)),
        ],
        out_specs=[
            pl.BlockSpec((BLK,), lambda i: (i,)),
            pl.BlockSpec((BLK,), lambda i: (i,)),
            pl.BlockSpec((BLK,), lambda i: (i,)),
            pl.BlockSpec((BLK,), lambda i: (i,)),
        ],
        out_shape=[
            jax.ShapeDtypeStruct((N_TOKENS,), jnp.float32),
            jax.ShapeDtypeStruct((N_TOKENS,), jnp.float32),
            jax.ShapeDtypeStruct((N_TOKENS,), jnp.int32),
            jax.ShapeDtypeStruct((N_TOKENS,), jnp.int32),
        ],
    )(h, weight, bias.reshape(N_EXP, 1))

    tw = jnp.stack([tw1, tw2], axis=-1)
    ti = jnp.stack([ti1, ti2], axis=-1)
    return (tw, ti)
